# baseline (device time: 6921 ns/iter reference)
import jax
import jax.numpy as jnp
from jax import lax
from jax.experimental import pallas as pl
from jax.experimental.pallas import tpu as pltpu

N_DEV = 16


def kernel(x):
    m, n = x.shape

    def body(x_hbm, out_ref, x_vmem, halo_ref, copy_sems, send_sems, recv_sems):
        my = lax.axis_index("i")
        left = (my - 1) % N_DEV
        right = (my + 1) % N_DEV
        has_left = my > 0
        has_right = my < N_DEV - 1

        in_copy = pltpu.make_async_copy(x_hbm, x_vmem, copy_sems.at[0])
        in_copy.start()

        barrier_sem = pltpu.get_barrier_semaphore()

        @pl.when(has_left)
        def _():
            pl.semaphore_signal(
                barrier_sem, inc=1,
                device_id=(left,), device_id_type=pl.DeviceIdType.MESH,
            )

        @pl.when(jnp.logical_not(has_left))
        def _():
            pl.semaphore_signal(barrier_sem, inc=1)

        @pl.when(has_right)
        def _():
            pl.semaphore_signal(
                barrier_sem, inc=1,
                device_id=(right,), device_id_type=pl.DeviceIdType.MESH,
            )

        @pl.when(jnp.logical_not(has_right))
        def _():
            pl.semaphore_signal(barrier_sem, inc=1)

        pl.semaphore_wait(barrier_sem, 2)
        in_copy.wait()

        send_last = pltpu.make_async_remote_copy(
            src_ref=x_vmem.at[pl.ds(m - 1, 1), :],
            dst_ref=halo_ref.at[0],
            send_sem=send_sems.at[0],
            recv_sem=recv_sems.at[0],
            device_id=(right,),
            device_id_type=pl.DeviceIdType.MESH,
        )
        send_first = pltpu.make_async_remote_copy(
            src_ref=x_vmem.at[pl.ds(0, 1), :],
            dst_ref=halo_ref.at[1],
            send_sem=send_sems.at[1],
            recv_sem=recv_sems.at[1],
            device_id=(left,),
            device_id_type=pl.DeviceIdType.MESH,
        )

        @pl.when(has_right)
        def _():
            send_last.start()

        @pl.when(has_left)
        def _():
            send_first.start()

        out_ref[pl.ds(1, m - 2), :] = (
            0.25 * x_vmem[pl.ds(0, m - 2), :]
            + 0.5 * x_vmem[pl.ds(1, m - 2), :]
            + 0.25 * x_vmem[pl.ds(2, m - 2), :]
        )

        @pl.when(has_left)
        def _():
            send_last.wait_recv()
            send_first.wait_send()

        @pl.when(has_right)
        def _():
            send_first.wait_recv()
            send_last.wait_send()

        row0 = (
            0.25 * halo_ref[0, :, :]
            + 0.5 * x_vmem[pl.ds(0, 1), :]
            + 0.25 * x_vmem[pl.ds(1, 1), :]
        )
        out_ref[pl.ds(0, 1), :] = jnp.where(my == 0, x_vmem[pl.ds(0, 1), :], row0)

        rowm = (
            0.25 * x_vmem[pl.ds(m - 2, 1), :]
            + 0.5 * x_vmem[pl.ds(m - 1, 1), :]
            + 0.25 * halo_ref[1, :, :]
        )
        out_ref[pl.ds(m - 1, 1), :] = jnp.where(
            my == N_DEV - 1, x_vmem[pl.ds(m - 1, 1), :], rowm
        )

    return pl.pallas_call(
        body,
        out_shape=jax.ShapeDtypeStruct((m, n), x.dtype),
        in_specs=[pl.BlockSpec(memory_space=pl.ANY)],
        out_specs=pl.BlockSpec(memory_space=pltpu.VMEM),
        scratch_shapes=[
            pltpu.VMEM((m, n), x.dtype),
            pltpu.VMEM((2, 1, n), x.dtype),
            pltpu.SemaphoreType.DMA((1,)),
            pltpu.SemaphoreType.DMA((2,)),
            pltpu.SemaphoreType.DMA((2,)),
        ],
        compiler_params=pltpu.CompilerParams(collective_id=0),
    )(x)


# device time: 6032 ns/iter; 1.1474x vs baseline; 1.1474x over previous
import jax
import jax.numpy as jnp
from jax import lax
from jax.experimental import pallas as pl
from jax.experimental.pallas import tpu as pltpu

N_DEV = 16
T = 128
ORDER = (1, 2, 0, 3)


def kernel(x):
    m, n = x.shape
    assert m == T * 4

    win = []
    for t in range(4):
        lo = max(0, T * t - 8)
        end = min(m, T * t + T + 8)
        win.append((lo, end - lo))
    max_ln = max(ln for _, ln in win)

    def body(
        x_hbm, out_hbm,
        xbuf, obuf, topbuf, botbuf, halo_ref,
        in_sems, out_sems, edge_sems, send_sems, recv_sems,
    ):
        my = lax.axis_index("i")
        left = (my - 1) % N_DEV
        right = (my + 1) % N_DEV
        has_left = my > 0
        has_right = my < N_DEV - 1

        def in_copy(k):
            t = ORDER[k]
            lo, ln = win[t]
            return pltpu.make_async_copy(
                x_hbm.at[pl.ds(lo, ln), :],
                xbuf.at[k % 2, pl.ds(0, ln), :],
                in_sems.at[k % 2],
            )

        def out_copy(k):
            t = ORDER[k]
            return pltpu.make_async_copy(
                obuf.at[k % 2],
                out_hbm.at[pl.ds(T * t, T), :],
                out_sems.at[k % 2],
            )

        in_copy(0).start()
        in_copy(1).start()
        top_copy = pltpu.make_async_copy(
            x_hbm.at[pl.ds(0, 8), :], topbuf, edge_sems.at[0]
        )
        bot_copy = pltpu.make_async_copy(
            x_hbm.at[pl.ds(m - 8, 8), :], botbuf, edge_sems.at[1]
        )
        top_copy.start()
        bot_copy.start()

        barrier_sem = pltpu.get_barrier_semaphore()

        @pl.when(has_left)
        def _():
            pl.semaphore_signal(
                barrier_sem, inc=1,
                device_id=(left,), device_id_type=pl.DeviceIdType.MESH,
            )

        @pl.when(jnp.logical_not(has_left))
        def _():
            pl.semaphore_signal(barrier_sem, inc=1)

        @pl.when(has_right)
        def _():
            pl.semaphore_signal(
                barrier_sem, inc=1,
                device_id=(right,), device_id_type=pl.DeviceIdType.MESH,
            )

        @pl.when(jnp.logical_not(has_right))
        def _():
            pl.semaphore_signal(barrier_sem, inc=1)

        pl.semaphore_wait(barrier_sem, 2)

        send_last = pltpu.make_async_remote_copy(
            src_ref=botbuf.at[pl.ds(7, 1), :],
            dst_ref=halo_ref.at[0],
            send_sem=send_sems.at[0],
            recv_sem=recv_sems.at[0],
            device_id=(right,),
            device_id_type=pl.DeviceIdType.MESH,
        )
        send_first = pltpu.make_async_remote_copy(
            src_ref=topbuf.at[pl.ds(0, 1), :],
            dst_ref=halo_ref.at[1],
            send_sem=send_sems.at[1],
            recv_sem=recv_sems.at[1],
            device_id=(left,),
            device_id_type=pl.DeviceIdType.MESH,
        )

        top_copy.wait()
        bot_copy.wait()

        @pl.when(has_right)
        def _():
            send_last.start()

        @pl.when(has_left)
        def _():
            send_first.start()

        for k, t in enumerate(ORDER):
            if k >= 2:
                out_copy(k - 2).wait()
            in_copy(k).wait()
            lo, _ = win[t]
            if t == 0:
                @pl.when(has_left)
                def _():
                    send_last.wait_recv()
                obuf[k % 2, pl.ds(1, T - 1), :] = (
                    0.25 * xbuf[k % 2, pl.ds(0, T - 1), :]
                    + 0.5 * xbuf[k % 2, pl.ds(1, T - 1), :]
                    + 0.25 * xbuf[k % 2, pl.ds(2, T - 1), :]
                )
                row0 = (
                    0.25 * halo_ref[0, :, :]
                    + 0.5 * xbuf[k % 2, pl.ds(0, 1), :]
                    + 0.25 * xbuf[k % 2, pl.ds(1, 1), :]
                )
                obuf[k % 2, pl.ds(0, 1), :] = jnp.where(
                    my == 0, xbuf[k % 2, pl.ds(0, 1), :], row0
                )
            elif t == 3:
                @pl.when(has_right)
                def _():
                    send_first.wait_recv()
                s = T * t - 1 - lo
                obuf[k % 2, pl.ds(0, T - 1), :] = (
                    0.25 * xbuf[k % 2, pl.ds(s, T - 1), :]
                    + 0.5 * xbuf[k % 2, pl.ds(s + 1, T - 1), :]
                    + 0.25 * xbuf[k % 2, pl.ds(s + 2, T - 1), :]
                )
                e = m - 1 - lo
                rowm = (
                    0.25 * xbuf[k % 2, pl.ds(e - 1, 1), :]
                    + 0.5 * xbuf[k % 2, pl.ds(e, 1), :]
                    + 0.25 * halo_ref[1, :, :]
                )
                obuf[k % 2, pl.ds(T - 1, 1), :] = jnp.where(
                    my == N_DEV - 1, xbuf[k % 2, pl.ds(e, 1), :], rowm
                )
            else:
                s = T * t - 1 - lo
                obuf[k % 2] = (
                    0.25 * xbuf[k % 2, pl.ds(s, T), :]
                    + 0.5 * xbuf[k % 2, pl.ds(s + 1, T), :]
                    + 0.25 * xbuf[k % 2, pl.ds(s + 2, T), :]
                )
            out_copy(k).start()
            if k + 2 < 4:
                in_copy(k + 2).start()

        @pl.when(has_left)
        def _():
            send_first.wait_send()

        @pl.when(has_right)
        def _():
            send_last.wait_send()

        out_copy(2).wait()
        out_copy(3).wait()

    return pl.pallas_call(
        body,
        out_shape=jax.ShapeDtypeStruct((m, n), x.dtype),
        in_specs=[pl.BlockSpec(memory_space=pl.ANY)],
        out_specs=pl.BlockSpec(memory_space=pl.ANY),
        scratch_shapes=[
            pltpu.VMEM((2, max_ln, n), x.dtype),
            pltpu.VMEM((2, T, n), x.dtype),
            pltpu.VMEM((8, n), x.dtype),
            pltpu.VMEM((8, n), x.dtype),
            pltpu.VMEM((2, 1, n), x.dtype),
            pltpu.SemaphoreType.DMA((2,)),
            pltpu.SemaphoreType.DMA((2,)),
            pltpu.SemaphoreType.DMA((2,)),
            pltpu.SemaphoreType.DMA((2,)),
            pltpu.SemaphoreType.DMA((2,)),
        ],
        compiler_params=pltpu.CompilerParams(collective_id=0),
    )(x)


# device time: 5950 ns/iter; 1.1632x vs baseline; 1.0138x over previous
import jax
import jax.numpy as jnp
from jax import lax
from jax.experimental import pallas as pl
from jax.experimental.pallas import tpu as pltpu

N_DEV = 16
E = 8


def kernel(x):
    m, n = x.shape

    mids = []
    for rlo in range(E, m - E, 128):
        c = min(128, m - E - rlo)
        lo = rlo - 8
        ln = min(m, rlo + c + 8) - lo
        mids.append((rlo, c, lo, ln))
    assert len(mids) == 4
    max_ln = max(t[3] for t in mids)

    def body(
        x_hbm, out_hbm,
        xbuf, obuf, topbuf, botbuf, ebuf, halo_ref,
        in_sems, out_sems, edge_sems, send_sems, recv_sems,
    ):
        my = lax.axis_index("i")
        left = (my - 1) % N_DEV
        right = (my + 1) % N_DEV
        has_left = my > 0
        has_right = my < N_DEV - 1

        def in_copy(k):
            rlo, c, lo, ln = mids[k]
            return pltpu.make_async_copy(
                x_hbm.at[pl.ds(lo, ln), :],
                xbuf.at[k % 2, pl.ds(0, ln), :],
                in_sems.at[k % 2],
            )

        def out_copy(k):
            rlo, c, lo, ln = mids[k]
            return pltpu.make_async_copy(
                obuf.at[k % 2, pl.ds(0, c), :],
                out_hbm.at[pl.ds(rlo, c), :],
                out_sems.at[k % 2],
            )

        in_copy(0).start()
        in_copy(1).start()
        top_copy = pltpu.make_async_copy(
            x_hbm.at[pl.ds(0, 2 * E), :], topbuf, edge_sems.at[0]
        )
        bot_copy = pltpu.make_async_copy(
            x_hbm.at[pl.ds(m - 2 * E, 2 * E), :], botbuf, edge_sems.at[1]
        )
        top_copy.start()
        bot_copy.start()

        barrier_sem = pltpu.get_barrier_semaphore()

        @pl.when(has_left)
        def _():
            pl.semaphore_signal(
                barrier_sem, inc=1,
                device_id=(left,), device_id_type=pl.DeviceIdType.MESH,
            )

        @pl.when(jnp.logical_not(has_left))
        def _():
            pl.semaphore_signal(barrier_sem, inc=1)

        @pl.when(has_right)
        def _():
            pl.semaphore_signal(
                barrier_sem, inc=1,
                device_id=(right,), device_id_type=pl.DeviceIdType.MESH,
            )

        @pl.when(jnp.logical_not(has_right))
        def _():
            pl.semaphore_signal(barrier_sem, inc=1)

        pl.semaphore_wait(barrier_sem, 2)

        send_last = pltpu.make_async_remote_copy(
            src_ref=botbuf.at[pl.ds(2 * E - 1, 1), :],
            dst_ref=halo_ref.at[0],
            send_sem=send_sems.at[0],
            recv_sem=recv_sems.at[0],
            device_id=(right,),
            device_id_type=pl.DeviceIdType.MESH,
        )
        send_first = pltpu.make_async_remote_copy(
            src_ref=topbuf.at[pl.ds(0, 1), :],
            dst_ref=halo_ref.at[1],
            send_sem=send_sems.at[1],
            recv_sem=recv_sems.at[1],
            device_id=(left,),
            device_id_type=pl.DeviceIdType.MESH,
        )

        top_copy.wait()
        bot_copy.wait()

        @pl.when(has_right)
        def _():
            send_last.start()

        @pl.when(has_left)
        def _():
            send_first.start()

        for k in range(4):
            if k >= 2:
                out_copy(k - 2).wait()
            in_copy(k).wait()
            rlo, c, lo, ln = mids[k]
            s = rlo - 1 - lo
            obuf[k % 2, pl.ds(0, c), :] = (
                0.25 * xbuf[k % 2, pl.ds(s, c), :]
                + 0.5 * xbuf[k % 2, pl.ds(s + 1, c), :]
                + 0.25 * xbuf[k % 2, pl.ds(s + 2, c), :]
            )
            out_copy(k).start()
            if k + 2 < 4:
                in_copy(k + 2).start()

        @pl.when(has_left)
        def _():
            send_last.wait_recv()

        ebuf[0, pl.ds(1, E - 1), :] = (
            0.25 * topbuf[pl.ds(0, E - 1), :]
            + 0.5 * topbuf[pl.ds(1, E - 1), :]
            + 0.25 * topbuf[pl.ds(2, E - 1), :]
        )
        row0 = (
            0.25 * halo_ref[0, :, :]
            + 0.5 * topbuf[pl.ds(0, 1), :]
            + 0.25 * topbuf[pl.ds(1, 1), :]
        )
        ebuf[0, pl.ds(0, 1), :] = jnp.where(
            my == 0, topbuf[pl.ds(0, 1), :], row0
        )
        etop_out = pltpu.make_async_copy(
            ebuf.at[0], out_hbm.at[pl.ds(0, E), :], edge_sems.at[0]
        )
        etop_out.start()

        @pl.when(has_right)
        def _():
            send_first.wait_recv()

        ebuf[1, pl.ds(0, E - 1), :] = (
            0.25 * botbuf[pl.ds(E - 1, E - 1), :]
            + 0.5 * botbuf[pl.ds(E, E - 1), :]
            + 0.25 * botbuf[pl.ds(E + 1, E - 1), :]
        )
        rowm = (
            0.25 * botbuf[pl.ds(2 * E - 2, 1), :]
            + 0.5 * botbuf[pl.ds(2 * E - 1, 1), :]
            + 0.25 * halo_ref[1, :, :]
        )
        ebuf[1, pl.ds(E - 1, 1), :] = jnp.where(
            my == N_DEV - 1, botbuf[pl.ds(2 * E - 1, 1), :], rowm
        )
        ebot_out = pltpu.make_async_copy(
            ebuf.at[1], out_hbm.at[pl.ds(m - E, E), :], edge_sems.at[1]
        )
        ebot_out.start()

        @pl.when(has_left)
        def _():
            send_first.wait_send()

        @pl.when(has_right)
        def _():
            send_last.wait_send()

        out_copy(2).wait()
        out_copy(3).wait()
        etop_out.wait()
        ebot_out.wait()

    return pl.pallas_call(
        body,
        out_shape=jax.ShapeDtypeStruct((m, n), x.dtype),
        in_specs=[pl.BlockSpec(memory_space=pl.ANY)],
        out_specs=pl.BlockSpec(memory_space=pl.ANY),
        scratch_shapes=[
            pltpu.VMEM((2, max_ln, n), x.dtype),
            pltpu.VMEM((2, 128, n), x.dtype),
            pltpu.VMEM((2 * E, n), x.dtype),
            pltpu.VMEM((2 * E, n), x.dtype),
            pltpu.VMEM((2, E, n), x.dtype),
            pltpu.VMEM((2, 1, n), x.dtype),
            pltpu.SemaphoreType.DMA((2,)),
            pltpu.SemaphoreType.DMA((2,)),
            pltpu.SemaphoreType.DMA((2,)),
            pltpu.SemaphoreType.DMA((2,)),
            pltpu.SemaphoreType.DMA((2,)),
        ],
        compiler_params=pltpu.CompilerParams(collective_id=0),
    )(x)


# device time: 5885 ns/iter; 1.1760x vs baseline; 1.0110x over previous
import jax
import jax.numpy as jnp
from jax import lax
from jax.experimental import pallas as pl
from jax.experimental.pallas import tpu as pltpu

N_DEV = 16
E = 8


def kernel(x):
    m, n = x.shape

    mids = []
    for rlo in range(E, m - E, 128):
        c = min(128, m - E - rlo)
        lo = rlo - 8
        ln = min(m, rlo + c + 8) - lo
        mids.append((rlo, c, lo, ln))
    assert len(mids) == 4
    max_ln = max(t[3] for t in mids)

    def body(
        x_hbm, out_hbm,
        xbuf, obuf, topbuf, botbuf, ebuf, halo_ref,
        in_sems, out_sems, edge_sems, send_sems, recv_sems,
    ):
        my = lax.axis_index("i")
        left = (my - 1) % N_DEV
        right = (my + 1) % N_DEV
        has_left = my > 0
        has_right = my < N_DEV - 1

        def in_copy(k):
            rlo, c, lo, ln = mids[k]
            return pltpu.make_async_copy(
                x_hbm.at[pl.ds(lo, ln), :],
                xbuf.at[k % 2, pl.ds(0, ln), :],
                in_sems.at[k % 2],
            )

        def out_copy(k):
            rlo, c, lo, ln = mids[k]
            return pltpu.make_async_copy(
                obuf.at[k % 2, pl.ds(0, c), :],
                out_hbm.at[pl.ds(rlo, c), :],
                out_sems.at[k % 2],
            )

        barrier_sem = pltpu.get_barrier_semaphore()

        @pl.when(has_left)
        def _():
            pl.semaphore_signal(
                barrier_sem, inc=1,
                device_id=(left,), device_id_type=pl.DeviceIdType.MESH,
            )

        @pl.when(jnp.logical_not(has_left))
        def _():
            pl.semaphore_signal(barrier_sem, inc=1)

        @pl.when(has_right)
        def _():
            pl.semaphore_signal(
                barrier_sem, inc=1,
                device_id=(right,), device_id_type=pl.DeviceIdType.MESH,
            )

        @pl.when(jnp.logical_not(has_right))
        def _():
            pl.semaphore_signal(barrier_sem, inc=1)

        top_copy = pltpu.make_async_copy(
            x_hbm.at[pl.ds(0, 2 * E), :], topbuf, edge_sems.at[0]
        )
        bot_copy = pltpu.make_async_copy(
            x_hbm.at[pl.ds(m - 2 * E, 2 * E), :], botbuf, edge_sems.at[1]
        )
        top_copy.start()
        bot_copy.start()
        in_copy(0).start()
        in_copy(1).start()

        pl.semaphore_wait(barrier_sem, 2)

        send_last = pltpu.make_async_remote_copy(
            src_ref=botbuf.at[pl.ds(2 * E - 1, 1), :],
            dst_ref=halo_ref.at[0],
            send_sem=send_sems.at[0],
            recv_sem=recv_sems.at[0],
            device_id=(right,),
            device_id_type=pl.DeviceIdType.MESH,
        )
        send_first = pltpu.make_async_remote_copy(
            src_ref=topbuf.at[pl.ds(0, 1), :],
            dst_ref=halo_ref.at[1],
            send_sem=send_sems.at[1],
            recv_sem=recv_sems.at[1],
            device_id=(left,),
            device_id_type=pl.DeviceIdType.MESH,
        )

        top_copy.wait()
        bot_copy.wait()

        @pl.when(has_right)
        def _():
            send_last.start()

        @pl.when(has_left)
        def _():
            send_first.start()

        for k in range(4):
            if k >= 2:
                out_copy(k - 2).wait()
            in_copy(k).wait()
            rlo, c, lo, ln = mids[k]
            s = rlo - 1 - lo
            obuf[k % 2, pl.ds(0, c), :] = (
                0.25 * xbuf[k % 2, pl.ds(s, c), :]
                + 0.5 * xbuf[k % 2, pl.ds(s + 1, c), :]
                + 0.25 * xbuf[k % 2, pl.ds(s + 2, c), :]
            )
            out_copy(k).start()
            if k + 2 < 4:
                in_copy(k + 2).start()

        ebuf[0, pl.ds(1, E - 1), :] = (
            0.25 * topbuf[pl.ds(0, E - 1), :]
            + 0.5 * topbuf[pl.ds(1, E - 1), :]
            + 0.25 * topbuf[pl.ds(2, E - 1), :]
        )
        ebuf[1, pl.ds(0, E - 1), :] = (
            0.25 * botbuf[pl.ds(E - 1, E - 1), :]
            + 0.5 * botbuf[pl.ds(E, E - 1), :]
            + 0.25 * botbuf[pl.ds(E + 1, E - 1), :]
        )

        @pl.when(has_left)
        def _():
            send_last.wait_recv()

        row0 = (
            0.25 * halo_ref[0, :, :]
            + 0.5 * topbuf[pl.ds(0, 1), :]
            + 0.25 * topbuf[pl.ds(1, 1), :]
        )
        ebuf[0, pl.ds(0, 1), :] = jnp.where(
            my == 0, topbuf[pl.ds(0, 1), :], row0
        )
        etop_out = pltpu.make_async_copy(
            ebuf.at[0], out_hbm.at[pl.ds(0, E), :], edge_sems.at[0]
        )
        etop_out.start()

        @pl.when(has_right)
        def _():
            send_first.wait_recv()

        rowm = (
            0.25 * botbuf[pl.ds(2 * E - 2, 1), :]
            + 0.5 * botbuf[pl.ds(2 * E - 1, 1), :]
            + 0.25 * halo_ref[1, :, :]
        )
        ebuf[1, pl.ds(E - 1, 1), :] = jnp.where(
            my == N_DEV - 1, botbuf[pl.ds(2 * E - 1, 1), :], rowm
        )
        ebot_out = pltpu.make_async_copy(
            ebuf.at[1], out_hbm.at[pl.ds(m - E, E), :], edge_sems.at[1]
        )
        ebot_out.start()

        @pl.when(has_left)
        def _():
            send_first.wait_send()

        @pl.when(has_right)
        def _():
            send_last.wait_send()

        out_copy(2).wait()
        out_copy(3).wait()
        etop_out.wait()
        ebot_out.wait()

    return pl.pallas_call(
        body,
        out_shape=jax.ShapeDtypeStruct((m, n), x.dtype),
        in_specs=[pl.BlockSpec(memory_space=pl.ANY)],
        out_specs=pl.BlockSpec(memory_space=pl.ANY),
        scratch_shapes=[
            pltpu.VMEM((2, max_ln, n), x.dtype),
            pltpu.VMEM((2, 128, n), x.dtype),
            pltpu.VMEM((2 * E, n), x.dtype),
            pltpu.VMEM((2 * E, n), x.dtype),
            pltpu.VMEM((2, E, n), x.dtype),
            pltpu.VMEM((2, 1, n), x.dtype),
            pltpu.SemaphoreType.DMA((2,)),
            pltpu.SemaphoreType.DMA((2,)),
            pltpu.SemaphoreType.DMA((2,)),
            pltpu.SemaphoreType.DMA((2,)),
            pltpu.SemaphoreType.DMA((2,)),
        ],
        compiler_params=pltpu.CompilerParams(collective_id=0),
    )(x)
